# baseline (speedup 1.0000x reference)
import functools

import jax
import jax.numpy as jnp
from jax import lax
from jax.experimental import pallas as pl
from jax.experimental.pallas import tpu as pltpu
from jax.experimental.pallas import tpu_sc as plsc

BATCH = 16384
INPUT_DIM = 39
N_CONT = 13
N_CAT = 26
EMB_DIM = 16
OUT_DIM = N_CONT + N_CAT * EMB_DIM

_NUM_CORES = 2
_NUM_SUBCORES = 16
NW = _NUM_CORES * _NUM_SUBCORES
ROWS_PW = BATCH // NW
CHUNK = 128
N_CHUNKS = ROWS_PW // CHUNK
LANES = 16


def _body(x_hbm, *args):
    tables = args[:N_CAT]
    out_hbm = args[N_CAT]
    x_v, idxs_v, embs_v, stage_v, sem = args[N_CAT + 1:]

    wid = lax.axis_index("s") * _NUM_CORES + lax.axis_index("c")
    base = wid * ROWS_PW

    iota = lax.iota(jnp.int32, LANES)

    for c in range(N_CHUNKS):
        cbase = base + c * CHUNK
        pltpu.sync_copy(x_hbm.at[pl.ds(cbase, CHUNK)], x_v)

        def extract(i, carry):
            rows = iota + i * LANES
            for j in range(INPUT_DIM):
                col = jnp.full((LANES,), j, jnp.int32)
                vals = plsc.load_gather(x_v, [rows, col])
                if j < N_CONT:
                    plsc.store_scatter(stage_v, [rows, col],
                                       vals.astype(jnp.float32))
                else:
                    t = jnp.full((LANES,), j - N_CONT, jnp.int32)
                    plsc.store_scatter(idxs_v, [t, rows], vals)
            return carry

        lax.fori_loop(0, CHUNK // LANES, extract, 0)

        for t in range(N_CAT):
            pltpu.make_async_copy(
                tables[t].at[idxs_v.at[t]], embs_v.at[t], sem).start()
        for t in range(N_CAT):
            pltpu.make_async_copy(
                tables[t].at[idxs_v.at[t]], embs_v.at[t], sem).wait()

        def assemble(r, carry):
            row = jnp.full((LANES,), r, jnp.int32)
            for t in range(N_CAT):
                v = embs_v[t, r]
                cols = iota + (N_CONT + t * EMB_DIM)
                plsc.store_scatter(stage_v, [row, cols], v)
            return carry

        lax.fori_loop(0, CHUNK, assemble, 0)

        pltpu.sync_copy(stage_v, out_hbm.at[pl.ds(cbase, CHUNK)])


_emb_kernel = functools.partial(
    pl.kernel,
    out_type=jax.ShapeDtypeStruct((BATCH, OUT_DIM), jnp.float32),
    mesh=plsc.VectorSubcoreMesh(core_axis_name="c", subcore_axis_name="s"),
    scratch_types=[
        pltpu.VMEM((CHUNK, INPUT_DIM), jnp.int32),
        pltpu.VMEM((N_CAT, CHUNK), jnp.int32),
        pltpu.VMEM((N_CAT, CHUNK, EMB_DIM), jnp.float32),
        pltpu.VMEM((CHUNK, OUT_DIM), jnp.float32),
        pltpu.SemaphoreType.DMA,
    ],
    compiler_params=pltpu.CompilerParams(use_tc_tiling_on_sc=False,
                                         needs_layout_passes=False),
)(_body)


def kernel(x, emb_0, emb_1, emb_2, emb_3, emb_4, emb_5, emb_6, emb_7,
           emb_8, emb_9, emb_10, emb_11, emb_12, emb_13, emb_14, emb_15,
           emb_16, emb_17, emb_18, emb_19, emb_20, emb_21, emb_22, emb_23,
           emb_24, emb_25):
    tables = (emb_0, emb_1, emb_2, emb_3, emb_4, emb_5, emb_6, emb_7,
              emb_8, emb_9, emb_10, emb_11, emb_12, emb_13, emb_14, emb_15,
              emb_16, emb_17, emb_18, emb_19, emb_20, emb_21, emb_22, emb_23,
              emb_24, emb_25)
    return _emb_kernel(x.astype(jnp.int32), *tables)
